# baseline (speedup 1.0000x reference)
import dataclasses

import jax
import jax.numpy as jnp
from jax import lax
from jax.experimental import pallas as pl
from jax.experimental.pallas import tpu as pltpu
from jax.experimental.pallas import tpu_sc as plsc

NUM_CLASSES = 1000
B0 = 4096
B1 = 20
N_WORKERS = 32
ROWS_PER_W = B0 // N_WORKERS
CH_D0 = 4
N_CHUNKS = ROWS_PER_W // CH_D0
CH_LOGICAL = CH_D0 * B1


def _sc_kernel(idx_hbm, out_hbm, buf, idx_v, sem):
    cid = lax.axis_index("c")
    sid = lax.axis_index("s")
    wid = sid * 2 + cid

    iota = lax.iota(jnp.int32, 16)
    zeros16 = jnp.zeros((16,), jnp.float32)
    ones16 = jnp.ones((16,), jnp.float32)

    def zero_row(r, carry):
        a = r // B1
        b = lax.rem(r, B1)
        for k in range(62):
            buf[a, b, pl.ds(k * 16, 16)] = zeros16
        buf[a, b, pl.ds(NUM_CLASSES - 16, 16)] = zeros16
        return carry

    lax.fori_loop(0, CH_LOGICAL, zero_row, 0)

    def scatter_chunk(value16):
        for k in range(5):
            r = iota + k * 16
            i0 = r // B1
            i1 = lax.rem(r, B1)
            i2 = idx_v[pl.ds(k * 16, 16)]
            plsc.store_scatter(buf, [i0, i1, i2], value16)

    def chunk_body(c, carry):
        d0 = wid * ROWS_PER_W + c * CH_D0
        lr = d0 * B1
        pltpu.sync_copy(idx_hbm.at[pl.ds(lr, CH_LOGICAL)], idx_v)
        scatter_chunk(ones16)
        pltpu.async_copy(buf, out_hbm.at[pl.ds(d0, CH_D0)], sem).wait()
        scatter_chunk(zeros16)
        return carry

    lax.fori_loop(0, N_CHUNKS, chunk_body, 0)


def kernel(inputs):
    idx = inputs.reshape(-1).astype(jnp.int32)
    mesh = plsc.VectorSubcoreMesh(core_axis_name="c", subcore_axis_name="s")
    cp = pltpu.CompilerParams()
    if "needs_layout_passes" in pltpu.CompilerParams.__dataclass_fields__:
        cp = dataclasses.replace(cp, needs_layout_passes=False)
    run = pl.kernel(
        _sc_kernel,
        mesh=mesh,
        compiler_params=cp,
        out_type=jax.ShapeDtypeStruct((B0, B1, NUM_CLASSES), jnp.float32),
        scratch_types=[
            pltpu.VMEM((CH_D0, B1, NUM_CLASSES), jnp.float32),
            pltpu.VMEM((CH_LOGICAL,), jnp.int32),
            pltpu.SemaphoreType.DMA,
        ],
    )
    return run(idx)


# speedup vs baseline: 1.0170x; 1.0170x over previous
import dataclasses

import jax
import jax.numpy as jnp
from jax import lax
from jax.experimental import pallas as pl
from jax.experimental.pallas import tpu as pltpu
from jax.experimental.pallas import tpu_sc as plsc

NUM_CLASSES = 1000
B0 = 4096
B1 = 20
N_WORKERS = 32
ROWS_PER_W = B0 // N_WORKERS
IDX_PER_W = ROWS_PER_W * B1
CH_D0 = 2
N_BUF = 2
N_CHUNKS = ROWS_PER_W // CH_D0
CH_LOGICAL = CH_D0 * B1
N_GROUPS = (CH_LOGICAL + 15) // 16


def _sc_kernel(idx_hbm, out_hbm, buf, idx_v, sems):
    cid = lax.axis_index("c")
    sid = lax.axis_index("s")
    wid = sid * 2 + cid

    iota = lax.iota(jnp.int32, 16)
    zeros16 = jnp.zeros((16,), jnp.float32)
    ones16 = jnp.ones((16,), jnp.float32)

    pltpu.sync_copy(idx_hbm.at[pl.ds(wid * IDX_PER_W, IDX_PER_W)], idx_v)

    def zero_row(r, carry):
        b = r // (CH_D0 * B1)
        a0 = lax.rem(r, CH_D0 * B1) // B1
        a1 = lax.rem(r, B1)
        for k in range(62):
            buf[b, a0, a1, pl.ds(k * 16, 16)] = zeros16
        buf[b, a0, a1, pl.ds(NUM_CLASSES - 16, 16)] = zeros16
        return carry

    lax.fori_loop(0, N_BUF * CH_LOGICAL, zero_row, 0)

    def scatter_chunk(b, c, value16):
        for k in range(N_GROUPS):
            g = iota + k * 16
            mask = g < CH_LOGICAL
            i0 = g // B1
            i1 = lax.rem(g, B1)
            i2 = idx_v[pl.ds(c * CH_LOGICAL + k * 16, 16)]
            plsc.store_scatter(buf.at[b], [i0, i1, i2], value16, mask=mask)

    def chunk_dma(b, c):
        d0 = wid * ROWS_PER_W + c * CH_D0
        return pltpu.make_async_copy(
            buf.at[b], out_hbm.at[pl.ds(d0, CH_D0)], sems.at[b])

    def chunk_body(c, carry):
        b = lax.rem(c, N_BUF)

        @pl.when(c >= N_BUF)
        def _reclaim():
            chunk_dma(b, c - N_BUF).wait()
            scatter_chunk(b, c - N_BUF, zeros16)

        scatter_chunk(b, c, ones16)
        chunk_dma(b, c).start()
        return carry

    lax.fori_loop(0, N_CHUNKS, chunk_body, 0)
    for j in range(N_BUF):
        c = N_CHUNKS - N_BUF + j
        chunk_dma(c % N_BUF, c).wait()


def kernel(inputs):
    idx = inputs.reshape(-1).astype(jnp.int32)
    mesh = plsc.VectorSubcoreMesh(core_axis_name="c", subcore_axis_name="s")
    cp = pltpu.CompilerParams()
    if "needs_layout_passes" in pltpu.CompilerParams.__dataclass_fields__:
        cp = dataclasses.replace(cp, needs_layout_passes=False)
    run = pl.kernel(
        _sc_kernel,
        mesh=mesh,
        compiler_params=cp,
        out_type=jax.ShapeDtypeStruct((B0, B1, NUM_CLASSES), jnp.float32),
        scratch_types=[
            pltpu.VMEM((N_BUF, CH_D0, B1, NUM_CLASSES), jnp.float32),
            pltpu.VMEM((IDX_PER_W,), jnp.int32),
            pltpu.SemaphoreType.DMA((N_BUF,)),
        ],
    )
    return run(idx)
